# baseline (speedup 1.0000x reference)
import functools

import jax
import jax.numpy as jnp
from jax import lax
from jax.experimental import pallas as pl
from jax.experimental.pallas import tpu as pltpu
from jax.experimental.pallas import tpu_sc as plsc

MULT = 8.0
BT = 128
NBUF = 4


def _make_sc_lookup(V, D, S0, S1):
    info = plsc.get_sparse_core_info()
    NC, NS, L = info.num_cores, info.num_subcores, info.num_lanes
    NW = NC * NS
    DT = D // 8
    assert D % L == 0 and S0 % (NW * BT) == 0
    nbt = S0 // BT
    bt_per_w = nbt // NW
    chunks = bt_per_w * S1
    assert chunks % NBUF == 0
    mesh = plsc.VectorSubcoreMesh(core_axis_name="c", subcore_axis_name="s")

    @functools.partial(
        pl.kernel,
        mesh=mesh,
        out_type=jax.ShapeDtypeStruct((S1, DT, nbt, 8, BT), jnp.float32),
        compiler_params=pltpu.CompilerParams(
            use_tc_tiling_on_sc=False, needs_layout_passes=False
        ),
        scratch_types=[
            pltpu.VMEM((S1, bt_per_w, BT), jnp.int32),
            pltpu.VMEM((NBUF, BT, D), jnp.float32),
            pltpu.VMEM((NBUF, DT, 8, BT), jnp.float32),
            pltpu.SemaphoreType.DMA,
            pltpu.SemaphoreType.DMA,
        ],
    )
    def k(table_hbm, idx_hbm, out_hbm, idx_all, rows_g, tiles_w, gsem, wsem):
        wid = lax.axis_index("s") * NC + lax.axis_index("c")
        bt0 = wid * bt_per_w

        pltpu.sync_copy(idx_hbm.at[:, pl.ds(bt0, bt_per_w)], idx_all)

        def fire(c, slot):
            s = lax.rem(c, S1)
            j = lax.div(c, S1)
            pltpu.async_copy(
                table_hbm.at[idx_all.at[s, j]], rows_g.at[slot], gsem
            )

        for b in range(NBUF):
            fire(b, b)

        def block_body(blk, carry):
            for b in range(NBUF):
                c = blk * NBUF + b
                pltpu.make_async_copy(
                    table_hbm.at[idx_all.at[0, 0]], rows_g.at[b], gsem
                ).wait()

                @pl.when(blk > 0)
                def _wait_wb():
                    pltpu.make_async_copy(
                        tiles_w.at[b], out_hbm.at[0, pl.ds(0, DT), 0], wsem
                    ).wait()

                def bg_body(bg, carry2):
                    rows16 = jnp.arange(L, dtype=jnp.int32) + bg * L
                    bsl = pl.ds(bg * L, L)
                    for d in range(D):
                        cols16 = jnp.full((L,), d, dtype=jnp.int32)
                        x = plsc.load_gather(rows_g.at[b], [rows16, cols16])
                        tiles_w[b, d // 8, d % 8, bsl] = x * MULT
                    return carry2

                lax.fori_loop(0, BT // L, bg_body, 0)

                s = lax.rem(c, S1)
                j = lax.div(c, S1)
                pltpu.async_copy(
                    tiles_w.at[b],
                    out_hbm.at[s, pl.ds(0, DT), bt0 + j],
                    wsem,
                )

                @pl.when(c + NBUF < chunks)
                def _refill():
                    fire(c + NBUF, b)

            return carry

        lax.fori_loop(0, chunks // NBUF, block_body, 0)

        for b in range(NBUF):
            pltpu.make_async_copy(
                tiles_w.at[b], out_hbm.at[0, pl.ds(0, DT), 0], wsem
            ).wait()

    def run(weight, input_ids):
        idxT = input_ids.astype(jnp.int32).T.reshape(S1, nbt, BT)
        out5 = k(weight, idxT)
        return out5.transpose(2, 4, 0, 1, 3).reshape(S0, S1, D)

    return run


def kernel(input_ids, weight):
    S0, S1 = input_ids.shape
    V, D = weight.shape
    lookup = _make_sc_lookup(V, D, S0, S1)
    return lookup(weight, input_ids)


# speedup vs baseline: 1.3252x; 1.3252x over previous
import functools

import jax
import jax.numpy as jnp
from jax import lax
from jax.experimental import pallas as pl
from jax.experimental.pallas import tpu as pltpu
from jax.experimental.pallas import tpu_sc as plsc

MULT = 8.0
BT = 128
NBUF = 4


def _make_sc_lookup(V, D, S0, S1):
    info = plsc.get_sparse_core_info()
    NC, NS, L = info.num_cores, info.num_subcores, info.num_lanes
    NW = NC * NS
    DT = D // 8
    assert D % L == 0 and S0 % (NW * BT) == 0
    nbt = S0 // BT
    bt_per_w = nbt // NW
    chunks = bt_per_w * S1
    assert chunks % NBUF == 0
    mesh = plsc.VectorSubcoreMesh(core_axis_name="c", subcore_axis_name="s")

    @functools.partial(
        pl.kernel,
        mesh=mesh,
        out_type=jax.ShapeDtypeStruct((S1, DT, nbt, 8, BT), jnp.float32),
        compiler_params=pltpu.CompilerParams(
            use_tc_tiling_on_sc=False, needs_layout_passes=False
        ),
        scratch_types=[
            pltpu.VMEM((S1, bt_per_w, BT), jnp.int32),
            pltpu.VMEM((NBUF, BT, D), jnp.float32),
            pltpu.VMEM((NBUF, DT, 8, BT), jnp.float32),
            pltpu.SemaphoreType.DMA,
            pltpu.SemaphoreType.DMA,
        ],
    )
    def k(table_hbm, idx_hbm, out_hbm, idx_all, rows_g, tiles_w, gsem, wsem):
        wid = lax.axis_index("s") * NC + lax.axis_index("c")
        bt0 = wid * bt_per_w

        pltpu.sync_copy(idx_hbm.at[:, pl.ds(bt0, bt_per_w)], idx_all)

        def fire(c, slot):
            s = lax.rem(c, S1)
            j = lax.div(c, S1)
            pltpu.async_copy(
                table_hbm.at[idx_all.at[s, j]], rows_g.at[slot], gsem
            )

        for b in range(NBUF):
            fire(b, b)

        def block_body(blk, carry):
            for b in range(NBUF):
                c = blk * NBUF + b
                pltpu.make_async_copy(
                    table_hbm.at[idx_all.at[0, 0]], rows_g.at[b], gsem
                ).wait()

                @pl.when(blk > 0)
                def _wait_wb():
                    pltpu.make_async_copy(
                        tiles_w.at[b], out_hbm.at[0, pl.ds(0, DT), 0], wsem
                    ).wait()

                @plsc.parallel_loop(0, BT // L, unroll=2)
                def bg_body(bg):
                    rows16 = jnp.arange(L, dtype=jnp.int32) + bg * L
                    bsl = pl.ds(bg * L, L)
                    for d in range(D):
                        cols16 = jnp.full((L,), d, dtype=jnp.int32)
                        x = plsc.load_gather(rows_g.at[b], [rows16, cols16])
                        tiles_w[b, d // 8, d % 8, bsl] = x * MULT

                s = lax.rem(c, S1)
                j = lax.div(c, S1)
                pltpu.async_copy(
                    tiles_w.at[b],
                    out_hbm.at[s, pl.ds(0, DT), bt0 + j],
                    wsem,
                )

                @pl.when(c + NBUF < chunks)
                def _refill():
                    fire(c + NBUF, b)

            return carry

        lax.fori_loop(0, chunks // NBUF, block_body, 0)

        for b in range(NBUF):
            pltpu.make_async_copy(
                tiles_w.at[b], out_hbm.at[0, pl.ds(0, DT), 0], wsem
            ).wait()

    def run(weight, input_ids):
        idxT = input_ids.astype(jnp.int32).T.reshape(S1, nbt, BT)
        out5 = k(weight, idxT)
        return out5.transpose(2, 4, 0, 1, 3).reshape(S0, S1, D)

    return run


def kernel(input_ids, weight):
    S0, S1 = input_ids.shape
    V, D = weight.shape
    lookup = _make_sc_lookup(V, D, S0, S1)
    return lookup(weight, input_ids)


# speedup vs baseline: 1.4220x; 1.0730x over previous
import functools

import jax
import jax.numpy as jnp
from jax import lax
from jax.experimental import pallas as pl
from jax.experimental.pallas import tpu as pltpu
from jax.experimental.pallas import tpu_sc as plsc

MULT = 8.0
BT = 128
NBUF = 2
WPAD = 128


def _make_sc_lookup(V, D, S0, S1):
    info = plsc.get_sparse_core_info()
    NC, NS, L = info.num_cores, info.num_subcores, info.num_lanes
    NW = NC * NS
    DT = D // 8
    assert D % L == 0 and S0 % (NW * BT) == 0
    nbt = S0 // BT
    bt_per_w = nbt // NW
    chunks = bt_per_w * S1
    assert chunks % NBUF == 0
    mesh = plsc.VectorSubcoreMesh(core_axis_name="c", subcore_axis_name="s")

    @functools.partial(
        pl.kernel,
        mesh=mesh,
        out_type=jax.ShapeDtypeStruct((S1, DT, nbt, 8, BT), jnp.float32),
        compiler_params=pltpu.CompilerParams(
            use_tc_tiling_on_sc=False, needs_layout_passes=False
        ),
        scratch_types=[
            pltpu.VMEM((S1, bt_per_w, BT), jnp.int32),
            pltpu.VMEM((NBUF, BT, WPAD), jnp.float32),
            pltpu.VMEM((NBUF, DT, 8, BT), jnp.float32),
            pltpu.SemaphoreType.DMA,
            pltpu.SemaphoreType.DMA,
        ],
    )
    def k(table_hbm, idx_hbm, out_hbm, idx_all, rows_g, tiles_w, gsem, wsem):
        wid = lax.axis_index("s") * NC + lax.axis_index("c")
        bt0 = wid * bt_per_w
        iota16 = jnp.arange(L, dtype=jnp.int32)

        pltpu.sync_copy(idx_hbm.at[:, pl.ds(bt0, bt_per_w)], idx_all)

        def fire(c, slot):
            s = lax.rem(c, S1)
            j = lax.div(c, S1)
            pltpu.async_copy(
                table_hbm.at[idx_all.at[s, j]], rows_g.at[slot], gsem
            )

        for b in range(NBUF):
            fire(b, b)

        def block_body(blk, carry):
            for b in range(NBUF):
                c = blk * NBUF + b
                pltpu.make_async_copy(
                    table_hbm.at[idx_all.at[0, 0]], rows_g.at[b], gsem
                ).wait()

                @pl.when(blk > 0)
                def _wait_wb():
                    pltpu.make_async_copy(
                        tiles_w.at[b], out_hbm.at[0, pl.ds(0, DT), 0], wsem
                    ).wait()

                @plsc.parallel_loop(0, BT // L, unroll=2)
                def bg_body(bg):
                    bivec = iota16 + bg * L
                    for j in range(L):
                        roll = iota16 ^ j
                        rq = lax.shift_right_logical(roll, 3)
                        rr = roll & 7
                        for dgrp in range(D // L):
                            d0 = dgrp * L
                            x = plsc.load_gather(
                                rows_g.at[b], [bivec, roll + d0]
                            )
                            plsc.store_scatter(
                                tiles_w.at[b],
                                [rq + 2 * dgrp, rr, bivec],
                                x * MULT,
                            )

                s = lax.rem(c, S1)
                j = lax.div(c, S1)
                pltpu.async_copy(
                    tiles_w.at[b],
                    out_hbm.at[s, pl.ds(0, DT), bt0 + j],
                    wsem,
                )

                @pl.when(c + NBUF < chunks)
                def _refill():
                    fire(c + NBUF, b)

            return carry

        lax.fori_loop(0, chunks // NBUF, block_body, 0)

        for b in range(NBUF):
            pltpu.make_async_copy(
                tiles_w.at[b], out_hbm.at[0, pl.ds(0, DT), 0], wsem
            ).wait()

    def run(weight, input_ids):
        wpad = jnp.pad(weight, ((0, 0), (0, WPAD - D)))
        idxT = input_ids.astype(jnp.int32).T.reshape(S1, nbt, BT)
        out5 = k(wpad, idxT)
        return out5.transpose(2, 4, 0, 1, 3).reshape(S0, S1, D)

    return run


def kernel(input_ids, weight):
    S0, S1 = input_ids.shape
    V, D = weight.shape
    lookup = _make_sc_lookup(V, D, S0, S1)
    return lookup(weight, input_ids)


# speedup vs baseline: 1.7686x; 1.2437x over previous
import functools

import jax
import jax.numpy as jnp
from jax import lax
from jax.experimental import pallas as pl
from jax.experimental.pallas import tpu as pltpu
from jax.experimental.pallas import tpu_sc as plsc

MULT = 8.0
BT = 128
NBUF = 2
WPAD = 128


def _make_sc_lookup(V, D, S0, S1):
    info = plsc.get_sparse_core_info()
    NC, NS, L = info.num_cores, info.num_subcores, info.num_lanes
    NW = NC * NS
    DT = D // 8
    assert D % L == 0 and S0 % (NW * BT) == 0
    nbt = S0 // BT
    bt_per_w = nbt // NW
    chunks = bt_per_w * S1
    assert chunks % NBUF == 0
    mesh = plsc.VectorSubcoreMesh(core_axis_name="c", subcore_axis_name="s")

    @functools.partial(
        pl.kernel,
        mesh=mesh,
        out_type=jax.ShapeDtypeStruct((S1, DT, nbt, 8, BT), jnp.float32),
        compiler_params=pltpu.CompilerParams(
            use_tc_tiling_on_sc=False, needs_layout_passes=False
        ),
        scratch_types=[
            pltpu.VMEM((S1, bt_per_w, BT), jnp.int32),
            pltpu.VMEM((NBUF, BT, WPAD), jnp.float32),
            pltpu.VMEM((NBUF, DT, 8, BT), jnp.float32),
            pltpu.SemaphoreType.DMA,
            pltpu.SemaphoreType.DMA,
        ],
    )
    def k(table_hbm, idx_hbm, out_hbm, idx_all, rows_g, tiles_w, gsem, wsem):
        wid = lax.axis_index("s") * NC + lax.axis_index("c")
        bt0 = wid * bt_per_w
        iota16 = jnp.arange(L, dtype=jnp.int32)

        pltpu.sync_copy(idx_hbm.at[:, pl.ds(bt0, bt_per_w)], idx_all)

        def fire(c, slot):
            s = lax.rem(c, S1)
            j = lax.div(c, S1)
            pltpu.async_copy(
                table_hbm.at[idx_all.at[s, j]], rows_g.at[slot], gsem
            )

        for b in range(NBUF):
            fire(b, b)

        def block_body(blk, carry):
            for b in range(NBUF):
                c = blk * NBUF + b
                pltpu.make_async_copy(
                    table_hbm.at[idx_all.at[0, 0]], rows_g.at[b], gsem
                ).wait()

                @pl.when(blk > 0)
                def _wait_wb():
                    pltpu.make_async_copy(
                        tiles_w.at[b], out_hbm.at[0, pl.ds(0, DT), 0], wsem
                    ).wait()

                @plsc.parallel_loop(0, BT // L, unroll=1)
                def bg_body(bg):
                    bivec = iota16 + bg * L
                    for j in range(L):
                        roll = iota16 ^ j
                        rq = lax.shift_right_logical(roll, 3)
                        rr = roll & 7
                        for dgrp in range(D // L):
                            d0 = dgrp * L
                            x = plsc.load_gather(
                                rows_g.at[b], [bivec, roll + d0]
                            )
                            plsc.store_scatter(
                                tiles_w.at[b],
                                [rq + 2 * dgrp, rr, bivec],
                                x * MULT,
                            )

                s = lax.rem(c, S1)
                j = lax.div(c, S1)
                pltpu.async_copy(
                    tiles_w.at[b],
                    out_hbm.at[s, pl.ds(0, DT), bt0 + j],
                    wsem,
                )

                @pl.when(c + NBUF < chunks)
                def _refill():
                    fire(c + NBUF, b)

            return carry

        lax.fori_loop(0, chunks // NBUF, block_body, 0)

        for b in range(NBUF):
            pltpu.make_async_copy(
                tiles_w.at[b], out_hbm.at[0, pl.ds(0, DT), 0], wsem
            ).wait()

    def run(weight, input_ids):
        wpad = jnp.pad(weight, ((0, 0), (0, WPAD - D)))
        idxT = input_ids.astype(jnp.int32).T.reshape(S1, nbt, BT)
        out5 = k(wpad, idxT)
        return out5.transpose(2, 4, 0, 1, 3).reshape(S0, S1, D)

    return run


def kernel(input_ids, weight):
    S0, S1 = input_ids.shape
    V, D = weight.shape
    lookup = _make_sc_lookup(V, D, S0, S1)
    return lookup(weight, input_ids)


# speedup vs baseline: 2.2136x; 1.2516x over previous
import functools

import jax
import jax.numpy as jnp
from jax import lax
from jax.experimental import pallas as pl
from jax.experimental.pallas import tpu as pltpu
from jax.experimental.pallas import tpu_sc as plsc

MULT = 8.0
BT = 128
NBUF = 2
WPAD = 128


def _make_sc_lookup(V, D, S0, S1):
    info = plsc.get_sparse_core_info()
    NC, NS, L = info.num_cores, info.num_subcores, info.num_lanes
    NW = NC * NS
    DT = D // 8
    assert D % L == 0 and S0 % (NW * BT) == 0
    nbt = S0 // BT
    bt_per_w = nbt // NW
    chunks = bt_per_w * S1
    assert chunks % NBUF == 0
    mesh = plsc.VectorSubcoreMesh(core_axis_name="c", subcore_axis_name="s")

    @functools.partial(
        pl.kernel,
        mesh=mesh,
        out_type=jax.ShapeDtypeStruct((S1, DT, nbt, 8, BT), jnp.float32),
        compiler_params=pltpu.CompilerParams(
            use_tc_tiling_on_sc=False, needs_layout_passes=False
        ),
        scratch_types=[
            pltpu.VMEM((S1, bt_per_w, BT), jnp.int32),
            pltpu.VMEM((NBUF, BT, WPAD), jnp.float32),
            pltpu.VMEM((NBUF, DT, 8, BT), jnp.float32),
            pltpu.SemaphoreType.DMA,
            pltpu.SemaphoreType.DMA,
        ],
    )
    def k(table_hbm, idx_hbm, out_hbm, idx_all, rows_g, tiles_w, gsem, wsem):
        wid = lax.axis_index("s") * NC + lax.axis_index("c")
        bt0 = wid * bt_per_w
        iota16 = jnp.arange(L, dtype=jnp.int32)

        pltpu.sync_copy(idx_hbm.at[:, pl.ds(bt0, bt_per_w)], idx_all)

        def fire(c, slot):
            s = lax.rem(c, S1)
            j = lax.div(c, S1)
            pltpu.async_copy(
                table_hbm.at[idx_all.at[s, j]], rows_g.at[slot], gsem
            )

        for b in range(NBUF):
            fire(b, b)

        def block_body(blk, carry):
            for b in range(NBUF):
                c = blk * NBUF + b
                pltpu.make_async_copy(
                    table_hbm.at[idx_all.at[0, 0]], rows_g.at[b], gsem
                ).wait()

                @pl.when(blk > 0)
                def _wait_wb():
                    pltpu.make_async_copy(
                        tiles_w.at[b], out_hbm.at[0, pl.ds(0, DT), 0], wsem
                    ).wait()

                @plsc.parallel_loop(0, L, unroll=1)
                def j_body(j):
                    roll = iota16 ^ j
                    rq = lax.shift_right_logical(roll, 3)
                    rr = roll & 7
                    for bg in range(BT // L):
                        bivec = iota16 + bg * L
                        for dgrp in range(D // L):
                            d0 = dgrp * L
                            x = plsc.load_gather(
                                rows_g.at[b], [bivec, roll + d0]
                            )
                            plsc.store_scatter(
                                tiles_w.at[b],
                                [rq + 2 * dgrp, rr, bivec],
                                x * MULT,
                            )

                s = lax.rem(c, S1)
                j = lax.div(c, S1)
                pltpu.async_copy(
                    tiles_w.at[b],
                    out_hbm.at[s, pl.ds(0, DT), bt0 + j],
                    wsem,
                )

                @pl.when(c + NBUF < chunks)
                def _refill():
                    fire(c + NBUF, b)

            return carry

        lax.fori_loop(0, chunks // NBUF, block_body, 0)

        for b in range(NBUF):
            pltpu.make_async_copy(
                tiles_w.at[b], out_hbm.at[0, pl.ds(0, DT), 0], wsem
            ).wait()

    def run(weight, input_ids):
        wpad = jnp.pad(weight, ((0, 0), (0, WPAD - D)))
        idxT = input_ids.astype(jnp.int32).T.reshape(S1, nbt, BT)
        out5 = k(wpad, idxT)
        return out5.transpose(2, 4, 0, 1, 3).reshape(S0, S1, D)

    return run


def kernel(input_ids, weight):
    S0, S1 = input_ids.shape
    V, D = weight.shape
    lookup = _make_sc_lookup(V, D, S0, S1)
    return lookup(weight, input_ids)
